# Initial kernel scaffold; baseline (speedup 1.0000x reference)
import jax
import jax.numpy as jnp
from jax.experimental import pallas as pl


def kernel(x, edge_index, batch_node_segment, W_self, W_neigh, b, gamma, beta, vn_emb, mlp_W1, mlp_b1, mlp_W2, mlp_b2):
    raise NotImplementedError("write your pallas kernel here")


# speedup vs baseline: 1.5821x; 1.5821x over previous
import functools

import jax
import jax.numpy as jnp
from jax import lax
from jax.experimental import pallas as pl
from jax.experimental.pallas import tpu as pltpu
from jax.experimental.pallas import tpu_sc as plsc

NUM_LAYERS = 3
D = 128
N = 10000
E = 320000
G = 10
GP = 16
EPS = 1e-5

NC = 2
NS = 16
NW = NC * NS
RPW = 313
BCH = 2000
NBCH = E // BCH
FLUSH = 2048
TRASH = FLUSH + 24
LROW = E + FLUSH + 32
ECH = 80
PCH = 64
NPAD = N + PCH


def _znodes(w):
    return jnp.where(w == NW - 1, N - (NW - 1) * RPW, RPW)


def _bucket_body(src_hbm, dst_hbm, srcl_hbm, dstl_hbm, cnt_hbm,
                 sstage, dstage, sbuf, dbuf, cntv, sem):
    cid = lax.axis_index("c")
    sid = lax.axis_index("s")
    w = sid * NC + cid
    lo = w * RPW
    hi = lo + RPW

    def _z(i, _):
        sbuf[pl.ds(i * 16, 16)] = jnp.zeros((16,), jnp.int32)
        dbuf[pl.ds(i * 16, 16)] = jnp.zeros((16,), jnp.int32)
        return _
    lax.fori_loop(0, (FLUSH + 32) // 16, _z, 0)

    def _chunk(c, carry):
        cur, base = carry
        pltpu.sync_copy(src_hbm.at[pl.ds(c * BCH, BCH)], sstage)
        pltpu.sync_copy(dst_hbm.at[pl.ds(c * BCH, BCH)], dstage)

        one = jnp.full((16,), 1, jnp.int32)

        def _vec(v, cur):
            dv = dstage[pl.ds(v * 16, 16)]
            sv = sstage[pl.ds(v * 16, 16)]
            a = dv - lo
            bb = dv - hi
            mi = (lax.shift_right_logical(bb, 31) & one) & (
                one - (lax.shift_right_logical(a, 31) & one))
            rank = plsc.cumsum(mi)
            pos = (cur + rank - 1) * mi + TRASH * (one - mi)
            plsc.store_scatter(sbuf, [pos], sv)
            plsc.store_scatter(dbuf, [pos], a)
            return cur + rank[15]
        cur = lax.fori_loop(0, BCH // 16, _vec, cur)

        k8 = cur & ~7
        basea = pl.multiple_of(base, 8)
        pltpu.sync_copy(sbuf.at[pl.ds(0, FLUSH)], srcl_hbm.at[w, pl.ds(basea, FLUSH)])
        pltpu.sync_copy(dbuf.at[pl.ds(0, FLUSH)], dstl_hbm.at[w, pl.ds(basea, FLUSH)])
        rem_s = sbuf[pl.ds(k8, 16)]
        rem_d = dbuf[pl.ds(k8, 16)]
        sbuf[pl.ds(0, 16)] = rem_s
        dbuf[pl.ds(0, 16)] = rem_d
        return cur - k8, base + k8

    cur, base = lax.fori_loop(0, NBCH, _chunk, (jnp.int32(0), jnp.int32(0)))
    basea = pl.multiple_of(base, 8)
    pltpu.sync_copy(sbuf.at[pl.ds(0, FLUSH)], srcl_hbm.at[w, pl.ds(basea, FLUSH)])
    pltpu.sync_copy(dbuf.at[pl.ds(0, FLUSH)], dstl_hbm.at[w, pl.ds(basea, FLUSH)])
    cntv[...] = jnp.full((16,), base + cur, jnp.int32)
    pltpu.sync_copy(cntv, cnt_hbm.at[w])


@functools.cache
def _bucket_kernel():
    return pl.kernel(
        _bucket_body,
        mesh=plsc.VectorSubcoreMesh(core_axis_name="c", subcore_axis_name="s",
                                    num_cores=NC, num_subcores=NS),
        out_type=(jax.ShapeDtypeStruct((NW, LROW), jnp.int32),
                  jax.ShapeDtypeStruct((NW, LROW), jnp.int32),
                  jax.ShapeDtypeStruct((NW, 16), jnp.int32)),
        compiler_params=pltpu.CompilerParams(use_tc_tiling_on_sc=False, needs_layout_passes=False),
        scratch_types=[
            pltpu.VMEM((BCH,), jnp.int32),
            pltpu.VMEM((BCH,), jnp.int32),
            pltpu.VMEM((FLUSH + 32,), jnp.int32),
            pltpu.VMEM((FLUSH + 32,), jnp.int32),
            pltpu.VMEM((16,), jnp.int32),
            pltpu.SemaphoreType.DMA,
        ],
    )


def _agg_body(msg_hbm, srcl_hbm, dstl_hbm, cnt_hbm, out_hbm,
              sidx, didx, rows_v, acc, cntv, sem):
    cid = lax.axis_index("c")
    sid = lax.axis_index("s")
    w = sid * NC + cid

    def _zrow(i, _):
        for k in range(8):
            acc[i, pl.ds(k * 16, 16)] = jnp.zeros((16,), jnp.float32)
        return _
    lax.fori_loop(0, RPW, _zrow, 0)

    pltpu.sync_copy(cnt_hbm.at[w], cntv)
    cnt = cntv[...][0]
    nch = (cnt + (ECH - 1)) // ECH

    def _chunk(c, carry2):
        pltpu.sync_copy(srcl_hbm.at[w, pl.ds(c * ECH, ECH)], sidx)
        pltpu.sync_copy(dstl_hbm.at[w, pl.ds(c * ECH, ECH)], didx.at[pl.ds(0, ECH)])
        pltpu.async_copy(msg_hbm.at[sidx], rows_v, sem).wait()

        def _edge(e, carry):
            @pl.when(c * ECH + e < cnt)
            def _do():
                dl = didx[pl.ds(e, 16)][0]
                for k in range(8):
                    v = rows_v[e, pl.ds(k * 16, 16)]
                    acc[dl, pl.ds(k * 16, 16)] = acc[dl, pl.ds(k * 16, 16)] + v
            return carry
        lax.fori_loop(0, ECH, _edge, 0)
        return carry2
    lax.fori_loop(0, nch, _chunk, 0)

    @pl.when(w < NW - 1)
    def _():
        pltpu.sync_copy(acc.at[pl.ds(0, RPW)], out_hbm.at[pl.ds(w * RPW, RPW)])

    @pl.when(w == NW - 1)
    def _():
        pltpu.sync_copy(acc.at[pl.ds(0, N - (NW - 1) * RPW)],
                        out_hbm.at[pl.ds((NW - 1) * RPW, N - (NW - 1) * RPW)])


@functools.cache
def _agg_kernel():
    return pl.kernel(
        _agg_body,
        mesh=plsc.VectorSubcoreMesh(core_axis_name="c", subcore_axis_name="s",
                                    num_cores=NC, num_subcores=NS),
        out_type=jax.ShapeDtypeStruct((N, D), jnp.float32),
        compiler_params=pltpu.CompilerParams(use_tc_tiling_on_sc=False, needs_layout_passes=False),
        scratch_types=[
            pltpu.VMEM((ECH,), jnp.int32),
            pltpu.VMEM((ECH + 16,), jnp.int32),
            pltpu.VMEM((ECH, D), jnp.float32),
            pltpu.VMEM((RPW, D), jnp.float32),
            pltpu.VMEM((16,), jnp.int32),
            pltpu.SemaphoreType.DMA,
        ],
    )


def _pool_body(h_hbm, seg_hbm, out_hbm, segv, buf, outv, sem):
    cid = lax.axis_index("c")
    sid = lax.axis_index("s")
    w = sid * NC + cid

    @pl.when(w < GP)
    def _():
        pltpu.sync_copy(seg_hbm, segv)

        one = jnp.full((16,), 1, jnp.int32)

        def _cnt(v, carry):
            lt, le = carry
            sv = segv[pl.ds(v * 16, 16)]
            lt_m = lax.shift_right_logical(sv - w, 31) & one
            le_m = lax.shift_right_logical(sv - (w + 1), 31) & one
            lt = lt + plsc.cumsum(lt_m)[15]
            le = le + plsc.cumsum(le_m)[15]
            return lt, le
        start, end = lax.fori_loop(0, N // 16, _cnt, (jnp.int32(0), jnp.int32(0)))

        nch = (end - start + (PCH - 1)) // PCH

        def _chunk(c, accs):
            pltpu.sync_copy(h_hbm.at[pl.ds(start + c * PCH, PCH)], buf)

            def _row(r, accs):
                def _add():
                    return tuple(accs[k] + buf[r, pl.ds(k * 16, 16)] for k in range(8))
                def _keep():
                    return accs
                return lax.cond(start + c * PCH + r < end, _add, _keep)
            return lax.fori_loop(0, PCH, _row, accs)

        accs = tuple(jnp.zeros((16,), jnp.float32) for _ in range(8))
        accs = lax.fori_loop(0, nch, _chunk, accs)
        for k in range(8):
            outv[pl.ds(k * 16, 16)] = accs[k]
        pltpu.sync_copy(outv, out_hbm.at[w])


@functools.cache
def _pool_kernel():
    return pl.kernel(
        _pool_body,
        mesh=plsc.VectorSubcoreMesh(core_axis_name="c", subcore_axis_name="s",
                                    num_cores=NC, num_subcores=NS),
        out_type=jax.ShapeDtypeStruct((GP, D), jnp.float32),
        compiler_params=pltpu.CompilerParams(use_tc_tiling_on_sc=False, needs_layout_passes=False),
        scratch_types=[
            pltpu.VMEM((N,), jnp.int32),
            pltpu.VMEM((PCH, D), jnp.float32),
            pltpu.VMEM((D,), jnp.float32),
            pltpu.SemaphoreType.DMA,
        ],
    )


def _bn(h, g_ref, be_ref):
    mean = jnp.sum(h, axis=0, keepdims=True) / N
    c = h - mean
    var = jnp.sum(c * c, axis=0, keepdims=True) / N
    return c / jnp.sqrt(var + EPS) * g_ref[:] + be_ref[:]


def _dense_vn_body(msg_ref, agg_ref, ws_ref, wn_ref, b_ref, g_ref, be_ref, out_ref):
    h = (jnp.dot(msg_ref[:], ws_ref[:], preferred_element_type=jnp.float32)
         + jnp.dot(agg_ref[:], wn_ref[:], preferred_element_type=jnp.float32)
         + b_ref[:])
    out_ref[pl.ds(0, N), :] = jax.nn.relu(_bn(h, g_ref, be_ref))
    out_ref[pl.ds(N, NPAD - N), :] = jnp.zeros((NPAD - N, D), jnp.float32)


def _dense_last_body(msg_ref, agg_ref, ws_ref, wn_ref, b_ref, g_ref, be_ref, out_ref):
    h = (jnp.dot(msg_ref[:], ws_ref[:], preferred_element_type=jnp.float32)
         + jnp.dot(agg_ref[:], wn_ref[:], preferred_element_type=jnp.float32)
         + b_ref[:])
    out_ref[:] = _bn(h, g_ref, be_ref)


def _mlp_bcast_body(h_ref, pooled_ref, vn_ref, w1_ref, b1_ref, w2_ref, b2_ref,
                    seg_ref, msg_ref, vnout_ref):
    pooled = pooled_ref[:] + vn_ref[:]
    hidden = jax.nn.relu(jnp.dot(pooled, w1_ref[:],
                                 preferred_element_type=jnp.float32) + b1_ref[:])
    vn = jnp.dot(hidden, w2_ref[:], preferred_element_type=jnp.float32) + b2_ref[:]
    vnout_ref[:] = vn
    h = h_ref[pl.ds(0, N), :]
    seg_b = jnp.broadcast_to(seg_ref[:], (N, D))
    msg = h
    for g in range(G):
        row = jnp.broadcast_to(vn[g:g + 1, :], (N, D))
        msg = msg + jnp.where(seg_b == g, row, jnp.float32(0.0))
    msg_ref[:] = msg


_dense_vn = pl.pallas_call(
    _dense_vn_body, out_shape=jax.ShapeDtypeStruct((NPAD, D), jnp.float32))
_dense_last = pl.pallas_call(
    _dense_last_body, out_shape=jax.ShapeDtypeStruct((N, D), jnp.float32))
_mlp_bcast = pl.pallas_call(
    _mlp_bcast_body, out_shape=(jax.ShapeDtypeStruct((N, D), jnp.float32),
                                jax.ShapeDtypeStruct((GP, D), jnp.float32)))


def kernel(x, edge_index, batch_node_segment, W_self, W_neigh, b, gamma, beta,
           vn_emb, mlp_W1, mlp_b1, mlp_W2, mlp_b2):
    src = edge_index[0].astype(jnp.int32)
    dst = edge_index[1].astype(jnp.int32)
    seg = batch_node_segment.astype(jnp.int32)
    seg2 = seg.reshape(N, 1)
    vnode = jnp.concatenate(
        [jnp.broadcast_to(vn_emb[0], (G, D)),
         jnp.zeros((GP - G, D), jnp.float32)], axis=0)

    srcl, dstl, cnts = _bucket_kernel()(src, dst)

    row = lambda a, l: a[l].reshape(1, -1)
    message = x
    for layer in range(NUM_LAYERS):
        agg = _agg_kernel()(message, srcl, dstl, cnts)
        if layer < NUM_LAYERS - 1:
            h = _dense_vn(message, agg, W_self[layer], W_neigh[layer],
                          row(b, layer), row(gamma, layer), row(beta, layer))
            pooled = _pool_kernel()(h, seg)
            message, vnode = _mlp_bcast(h, pooled, vnode,
                                        mlp_W1[layer], row(mlp_b1, layer),
                                        mlp_W2[layer], row(mlp_b2, layer), seg2)
        else:
            message = _dense_last(message, agg, W_self[layer], W_neigh[layer],
                                  row(b, layer), row(gamma, layer), row(beta, layer))
    return message
